# baseline (speedup 1.0000x reference)
import functools

import jax
import jax.numpy as jnp
from jax import lax
from jax.experimental import pallas as pl
from jax.experimental.pallas import tpu as pltpu
from jax.experimental.pallas import tpu_sc as plsc

_INFO = plsc.get_sparse_core_info()
_NC, _NS = _INFO.num_cores, _INFO.num_subcores
_NW = _NC * _NS

_B = 16384 * 200
_D = 64
_PER_W = _B // _NW
_CHUNK = 800
_N_CHUNKS = _PER_W // _CHUNK


def _embed_kernel(x_hbm, table_hbm, out_hbm, idx_v, rows_v, sem):
    wid = lax.axis_index("s") * _NC + lax.axis_index("c")
    base = wid * _PER_W

    def body(g, carry):
        off = base + g * _CHUNK
        pltpu.sync_copy(x_hbm.at[pl.ds(off, _CHUNK)], idx_v)
        pltpu.async_copy(table_hbm.at[idx_v], rows_v, sem).wait()
        pltpu.sync_copy(rows_v, out_hbm.at[pl.ds(off, _CHUNK)])
        return carry

    lax.fori_loop(0, _N_CHUNKS, body, 0)


@jax.jit
def kernel(x, table):
    xf = x.reshape(_B)
    mesh = plsc.VectorSubcoreMesh(core_axis_name="c", subcore_axis_name="s")
    run = functools.partial(
        pl.kernel,
        mesh=mesh,
        out_type=jax.ShapeDtypeStruct((_B, _D), jnp.float32),
        scratch_types=[
            pltpu.VMEM((_CHUNK,), jnp.int32),
            pltpu.VMEM((_CHUNK, _D), jnp.float32),
            pltpu.SemaphoreType.DMA,
        ],
        compiler_params=pltpu.CompilerParams(use_tc_tiling_on_sc=False),
    )(_embed_kernel)
    out = run(xf, table)
    return out.reshape(16384, 200, _D)


# speedup vs baseline: 5.3505x; 5.3505x over previous
import functools

import jax
import jax.numpy as jnp
from jax import lax
from jax.experimental import pallas as pl
from jax.experimental.pallas import tpu as pltpu
from jax.experimental.pallas import tpu_sc as plsc

_INFO = plsc.get_sparse_core_info()
_NC, _NS = _INFO.num_cores, _INFO.num_subcores
_NW = _NC * _NS

_B = 16384 * 200
_D = 64
_PER_W = _B // _NW
_CHUNK = 800
_N_CHUNKS = _PER_W // _CHUNK
_ROW_UNROLL = 4


def _embed_kernel(x_hbm, table_hbm, out_hbm, table_v, idx_v, rows_v,
                  outsem0, outsem1):
    wid = lax.axis_index("s") * _NC + lax.axis_index("c")
    base = wid * _PER_W
    pltpu.sync_copy(table_hbm, table_v)

    def assemble(rows_ref):
        def body(i, carry):
            vec = idx_v[pl.ds(i * 16, 16)]
            for u in range(16):
                row = i * 16 + u
                s = vec[u]
                for g in range(_D // 16):
                    rows_ref[row, pl.ds(16 * g, 16)] = (
                        table_v[s, pl.ds(16 * g, 16)])
            return carry
        lax.fori_loop(0, _CHUNK // 16, body, 0)

    def chunk_step(j, b, sem):
        g = j * 2 + b
        off = base + g * _CHUNK
        rows_ref = rows_v.at[b]
        pltpu.sync_copy(x_hbm.at[pl.ds(off, _CHUNK)], idx_v)

        @pl.when(j >= 1)
        def _():
            pltpu.make_async_copy(
                rows_ref, out_hbm.at[pl.ds(off, _CHUNK)], sem).wait()

        assemble(rows_ref)
        pltpu.async_copy(rows_ref, out_hbm.at[pl.ds(off, _CHUNK)], sem)

    def outer(j, carry):
        chunk_step(j, 0, outsem0)
        chunk_step(j, 1, outsem1)
        return carry

    lax.fori_loop(0, _N_CHUNKS // 2, outer, 0)

    pltpu.make_async_copy(
        rows_v.at[0], out_hbm.at[pl.ds(base, _CHUNK)], outsem0).wait()
    pltpu.make_async_copy(
        rows_v.at[1], out_hbm.at[pl.ds(base, _CHUNK)], outsem1).wait()


@jax.jit
def kernel(x, table):
    xf = x.reshape(_B)
    mesh = plsc.VectorSubcoreMesh(core_axis_name="c", subcore_axis_name="s")
    run = functools.partial(
        pl.kernel,
        mesh=mesh,
        out_type=jax.ShapeDtypeStruct((_B, _D), jnp.float32),
        scratch_types=[
            pltpu.VMEM((8, _D), jnp.float32),
            pltpu.VMEM((_CHUNK,), jnp.int32),
            pltpu.VMEM((2, _CHUNK, _D), jnp.float32),
            pltpu.SemaphoreType.DMA,
            pltpu.SemaphoreType.DMA,
        ],
        compiler_params=pltpu.CompilerParams(use_tc_tiling_on_sc=False),
    )(_embed_kernel)
    out = run(xf, table)
    return out.reshape(16384, 200, _D)


# speedup vs baseline: 7.2635x; 1.3575x over previous
import functools

import jax
import jax.numpy as jnp
from jax import lax
from jax.experimental import pallas as pl
from jax.experimental.pallas import tpu as pltpu
from jax.experimental.pallas import tpu_sc as plsc

_INFO = plsc.get_sparse_core_info()
_NC, _NS = _INFO.num_cores, _INFO.num_subcores
_NW = _NC * _NS

_B = 16384 * 200
_D = 64
_PER_W = _B // _NW
_CHUNK = 800
_N_CHUNKS = _PER_W // _CHUNK
_ROW_UNROLL = 4


def _embed_kernel(x_hbm, table_hbm, out_hbm, table_v, idx_v, rows_v,
                  outsem0, outsem1):
    wid = lax.axis_index("s") * _NC + lax.axis_index("c")
    base = wid * _PER_W
    pltpu.sync_copy(table_hbm, table_v)

    def assemble(rows_ref):
        def body(i, carry):
            vec = idx_v[pl.ds(i * 16, 16)]
            for half in range(2):
                loads = []
                for u in range(8):
                    s = vec[half * 8 + u]
                    loads.append(
                        [table_v[s, pl.ds(16 * g, 16)]
                         for g in range(_D // 16)])
                for u in range(8):
                    row = i * 16 + half * 8 + u
                    for g in range(_D // 16):
                        rows_ref[row, pl.ds(16 * g, 16)] = loads[u][g]
            return carry
        lax.fori_loop(0, _CHUNK // 16, body, 0)

    def chunk_step(j, b, sem):
        g = j * 2 + b
        off = base + g * _CHUNK
        rows_ref = rows_v.at[b]
        pltpu.sync_copy(x_hbm.at[pl.ds(off, _CHUNK)], idx_v)

        @pl.when(j >= 1)
        def _():
            pltpu.make_async_copy(
                rows_ref, out_hbm.at[pl.ds(off, _CHUNK)], sem).wait()

        assemble(rows_ref)
        pltpu.async_copy(rows_ref, out_hbm.at[pl.ds(off, _CHUNK)], sem)

    def outer(j, carry):
        chunk_step(j, 0, outsem0)
        chunk_step(j, 1, outsem1)
        return carry

    lax.fori_loop(0, _N_CHUNKS // 2, outer, 0)

    pltpu.make_async_copy(
        rows_v.at[0], out_hbm.at[pl.ds(base, _CHUNK)], outsem0).wait()
    pltpu.make_async_copy(
        rows_v.at[1], out_hbm.at[pl.ds(base, _CHUNK)], outsem1).wait()


@jax.jit
def kernel(x, table):
    xf = x.reshape(_B)
    mesh = plsc.VectorSubcoreMesh(core_axis_name="c", subcore_axis_name="s")
    run = functools.partial(
        pl.kernel,
        mesh=mesh,
        out_type=jax.ShapeDtypeStruct((_B, _D), jnp.float32),
        scratch_types=[
            pltpu.VMEM((8, _D), jnp.float32),
            pltpu.VMEM((_CHUNK,), jnp.int32),
            pltpu.VMEM((2, _CHUNK, _D), jnp.float32),
            pltpu.SemaphoreType.DMA,
            pltpu.SemaphoreType.DMA,
        ],
        compiler_params=pltpu.CompilerParams(use_tc_tiling_on_sc=False),
    )(_embed_kernel)
    out = run(xf, table)
    return out.reshape(16384, 200, _D)
